# Initial kernel scaffold; baseline (speedup 1.0000x reference)
import jax
import jax.numpy as jnp
from jax.experimental import pallas as pl


def kernel(atom_feats, bond_feats, global_feats, atom_segment_ids, bond_segment_ids, W_atom, b_atom, W_bond, b_bond):
    raise NotImplementedError("write your pallas kernel here")


# speedup vs baseline: 7.4155x; 7.4155x over previous
import functools

import jax
import jax.numpy as jnp
from jax import lax
from jax.experimental import pallas as pl
from jax.experimental.pallas import tpu as pltpu
from jax.experimental.pallas import tpu_sc as plsc

NC = 2
NS = 16
NW = NC * NS
L = 16
B = 256
F = 256
FC = F // L
RB = 32
NEG = -1e30

NA = 50000
NB = 100000
CHA = 1568
CHB = 3136

assert CHA * NW >= NA and CHB * NW >= NB
assert CHA % 8 == 0 and CHB % 8 == 0
assert (NA - RB) % 8 == 0 and (NB - RB) % 8 == 0


def _sload_i32(ref, idx):
    return ref[pl.ds(idx, L)][0]


def _do_ntype(x_hbm, seg_hbm, w_hbm, b_hbm, N, CH,
              acc_o, m_o, s_o,
              acc_ref, m_ref, s_ref, seg0, segb, xbuf, wbuf, bbuf,
              sem0, sem1, sem0s, sem1s, wid):
    start = wid * CH
    end = jnp.minimum(start + CH, N)
    nrows = end - start

    pltpu.sync_copy(w_hbm, wbuf)
    pltpu.sync_copy(b_hbm, bbuf)
    pltpu.sync_copy(seg_hbm.at[pl.ds(start, L)], seg0)

    negv = jnp.full((L,), NEG, jnp.float32)
    zerov = jnp.zeros((L,), jnp.float32)

    def _init(i, c):
        m_ref[pl.ds(i * L, L)] = negv
        return c
    lax.fori_loop(0, B, _init, 0)

    wk = [wbuf[pl.ds(k * L, L)] for k in range(FC)]
    bvec = bbuf[:]

    nblk = (nrows + RB - 1) // RB

    def blk_row0(bi):
        return jnp.minimum(start + bi * RB, N - RB)

    def start_dma(bi, xslot, sslot, sem, sems):
        pltpu.async_copy(x_hbm.at[pl.ds(blk_row0(bi), RB)], xslot, sem)
        pltpu.async_copy(seg_hbm.at[pl.ds(blk_row0(bi), RB)],
                         sslot.at[pl.ds(0, RB)], sems)

    def wait_dma(bi, xslot, sslot, sem, sems):
        pltpu.make_async_copy(x_hbm.at[pl.ds(blk_row0(bi), RB)], xslot,
                              sem).wait()
        pltpu.make_async_copy(seg_hbm.at[pl.ds(blk_row0(bi), RB)],
                              sslot.at[pl.ds(0, RB)], sems).wait()

    def flush(cur, m, s, accs):
        for k in range(FC):
            acc_ref[cur, pl.ds(k * L, L)] = accs[k]
        m_ref[pl.ds(cur * L, L)] = m
        s_ref[pl.ds(cur * L, L)] = s

    def process_block(bi, carry, xslot, sslot):
        gstart = start + bi * RB
        row0 = blk_row0(bi)
        cnt = jnp.minimum(RB, end - gstart)
        base_off = gstart - row0

        def body(r, carry):
            cur, m, s = carry[0], carry[1], carry[2]
            accs = list(carry[3:])
            xoff = base_off + r
            xk = [xslot[xoff, pl.ds(k * L, L)] for k in range(FC)]
            p = xk[0] * wk[0]
            for k in range(1, FC):
                p = p + xk[k] * wk[k]
            tot = jnp.sum(p)
            gv = jnp.broadcast_to(tot, (L,)) + bvec
            gv = jnp.where(gv >= 0.0, gv, gv * 0.01)

            sg = _sload_i32(sslot, xoff)
            changed = sg != cur

            @pl.when(changed)
            def _():
                flush(cur, m, s, accs)

            m_eff = jnp.where(changed, negv, m)
            m_new = jnp.maximum(m_eff, gv)
            factor = jnp.exp(m_eff - m_new)
            wt = jnp.exp(gv - m_new)
            s_new = s * factor + wt
            new_accs = [accs[k] * factor + wt * xk[k] for k in range(FC)]
            return (sg, m_new, s_new) + tuple(new_accs)

        return lax.fori_loop(0, cnt, body, carry)

    start_dma(0, xbuf.at[0], segb.at[0], sem0, sem0s)

    @pl.when(nblk > 1)
    def _():
        start_dma(1, xbuf.at[1], segb.at[1], sem1, sem1s)

    init_seg = seg0[:][0]
    carry = (init_seg, negv, zerov) + tuple(zerov for _ in range(FC))

    npairs = (nblk + 1) // 2

    def pair_body(p, carry):
        b0 = 2 * p
        wait_dma(b0, xbuf.at[0], segb.at[0], sem0, sem0s)
        carry = process_block(b0, carry, xbuf.at[0], segb.at[0])

        @pl.when(b0 + 2 < nblk)
        def _():
            start_dma(b0 + 2, xbuf.at[0], segb.at[0], sem0, sem0s)

        def second(carry):
            wait_dma(b0 + 1, xbuf.at[1], segb.at[1], sem1, sem1s)
            carry = process_block(b0 + 1, carry, xbuf.at[1], segb.at[1])

            @pl.when(b0 + 3 < nblk)
            def _():
                start_dma(b0 + 3, xbuf.at[1], segb.at[1], sem1, sem1s)
            return carry

        return lax.cond(b0 + 1 < nblk, second, lambda c: c, carry)

    carry = lax.fori_loop(0, npairs, pair_body, carry)

    flush(carry[0], carry[1], carry[2], list(carry[3:]))

    pltpu.sync_copy(acc_ref, acc_o.at[wid])
    pltpu.sync_copy(m_ref, m_o.at[wid])
    pltpu.sync_copy(s_ref, s_o.at[wid])


def _sc_body(atomX, bondX, segA, segB, wA, bA, wB, bB,
             accA_o, mA_o, sA_o, accB_o, mB_o, sB_o,
             acc_ref, m_ref, s_ref, seg0, segb, xbuf, wbuf, bbuf,
             sem0, sem1, sem0s, sem1s):
    wid = lax.axis_index("s") * NC + lax.axis_index("c")
    args = (acc_ref, m_ref, s_ref, seg0, segb, xbuf, wbuf, bbuf,
            sem0, sem1, sem0s, sem1s, wid)
    _do_ntype(atomX, segA, wA, bA, NA, CHA, accA_o, mA_o, sA_o, *args)
    _do_ntype(bondX, segB, wB, bB, NB, CHB, accB_o, mB_o, sB_o, *args)


@jax.jit
def _sc_stage(atom_feats, bond_feats, segA, segB, wA, bA16, wB, bB16):
    mesh = plsc.VectorSubcoreMesh(core_axis_name="c", subcore_axis_name="s")
    f32 = jnp.float32
    out_type = (
        jax.ShapeDtypeStruct((NW, B, F), f32),
        jax.ShapeDtypeStruct((NW, B * L), f32),
        jax.ShapeDtypeStruct((NW, B * L), f32),
        jax.ShapeDtypeStruct((NW, B, F), f32),
        jax.ShapeDtypeStruct((NW, B * L), f32),
        jax.ShapeDtypeStruct((NW, B * L), f32),
    )
    scratch = [
        pltpu.VMEM((B, F), f32),
        pltpu.VMEM((B * L,), f32),
        pltpu.VMEM((B * L,), f32),
        pltpu.VMEM((L,), jnp.int32),
        pltpu.VMEM((2, RB + L), jnp.int32),
        pltpu.VMEM((2, RB, F), f32),
        pltpu.VMEM((F,), f32),
        pltpu.VMEM((L,), f32),
        pltpu.SemaphoreType.DMA,
        pltpu.SemaphoreType.DMA,
        pltpu.SemaphoreType.DMA,
        pltpu.SemaphoreType.DMA,
    ]
    return pl.kernel(
        _sc_body, out_type=out_type, mesh=mesh, scratch_types=scratch,
        compiler_params=pltpu.CompilerParams(needs_layout_passes=False),
    )(atom_feats, bond_feats, segA, segB, wA, bA16, wB, bB16)


def _combine_body(accA, mA, sA, accB, mB, sB, gf, out_ref):
    def readout(acc_ref, m_ref, s_ref):
        m = m_ref[...]
        s = s_ref[...]
        valid = m > -1e29
        mg = jnp.max(jnp.where(valid, m, NEG), axis=1, keepdims=True)
        e = jnp.where(valid, jnp.exp(m - mg), 0.0)
        gsum = jnp.sum(jnp.where(valid, s, 0.0) * e, axis=1,
                       keepdims=True)
        inv = jnp.where(gsum > 0.0, 1.0 / gsum, 0.0)
        scale = e * inv
        out = jnp.zeros((B, F), jnp.float32)
        for w in range(NW):
            contrib = acc_ref[w] * scale[:, w:w + 1]
            out = out + jnp.where(valid[:, w:w + 1], contrib, 0.0)
        return out

    out_ref[:, 0:F] = readout(accA, mA, sA)
    out_ref[:, F:2 * F] = readout(accB, mB, sB)
    out_ref[:, 2 * F:] = gf[...]


def kernel(atom_feats, bond_feats, global_feats, atom_segment_ids,
           bond_segment_ids, W_atom, b_atom, W_bond, b_bond):
    f32 = jnp.float32
    wA = W_atom[:, 0].astype(f32)
    wB = W_bond[:, 0].astype(f32)
    bA16 = jnp.broadcast_to(b_atom.astype(f32), (L,))
    bB16 = jnp.broadcast_to(b_bond.astype(f32), (L,))
    segA = atom_segment_ids.astype(jnp.int32)
    segB = bond_segment_ids.astype(jnp.int32)

    accA, mA, sA, accB, mB, sB = _sc_stage(
        atom_feats.astype(f32), bond_feats.astype(f32),
        segA, segB, wA, bA16, wB, bB16)

    out = pl.pallas_call(
        _combine_body,
        out_shape=jax.ShapeDtypeStruct((B, 2 * F + 128), f32),
    )(accA, mA.reshape(NW, B, L)[:, :, 0].T, sA.reshape(NW, B, L)[:, :, 0].T,
      accB, mB.reshape(NW, B, L)[:, :, 0].T, sB.reshape(NW, B, L)[:, :, 0].T,
      global_feats.astype(f32))
    return out
